# baseline (speedup 1.0000x reference)
import jax
import jax.numpy as jnp
from jax import lax
from jax.experimental import pallas as pl
from jax.experimental.pallas import tpu as pltpu
from jax.experimental.pallas import tpu_sc as plsc

IPT_VOCAB = 100000
EMBED = 1024
NUM_TOK = 4 * 4096
LANES = 16

NC, NS = 2, 16
NW = NC * NS
T_PER_W = NUM_TOK // NW
CHUNK = 8
NCHUNK = T_PER_W // CHUNK
NSETS = 4
AHEAD = 2
ZROW = 1026
NZ = 512


def _sc_body(tok_hbm, posid_hbm, word_hbm, post_hbm, small_hbm, out_hbm,
             tok_v, pos_v, widx_v, pidx_v, sidx_v, flag_s,
             *bufs_and_sems):
    bufs = tuple((bufs_and_sems[3 * k], bufs_and_sems[3 * k + 1],
                  bufs_and_sems[3 * k + 2]) for k in range(NSETS))
    sem_g = bufs_and_sems[3 * NSETS:3 * NSETS + NSETS]
    sem_s = bufs_and_sems[3 * NSETS + NSETS:]

    wid = lax.axis_index("s") * NC + lax.axis_index("c")
    base = wid * T_PER_W

    pltpu.sync_copy(tok_hbm.at[pl.ds(base, T_PER_W)], tok_v)
    pltpu.sync_copy(posid_hbm.at[pl.ds(base, T_PER_W)], pos_v)

    def idx_body(i, _):
        sl = pl.ds(i * LANES, LANES)
        t = tok_v[sl]
        p = pos_v[sl]
        ipt = t < IPT_VOCAB
        zrow = ZROW + (i % (NZ // LANES)) * LANES + lax.iota(jnp.int32, 16)
        nonipt_row = t - IPT_VOCAB
        widx_v[sl] = jnp.where(ipt, t, nonipt_row)
        pidx_v[sl] = jnp.where(ipt, p, nonipt_row)
        sidx_v[sl] = jnp.where(ipt, zrow, nonipt_row)
        m = t[0]
        for k in range(1, LANES):
            m = jnp.maximum(m, t[k])
        flag_s[i] = m
        return _

    lax.fori_loop(0, T_PER_W // LANES, idx_body, None)

    def needs_small(c):
        return flag_s[c * CHUNK // LANES] >= IPT_VOCAB

    def issue_gathers(c, st):
        isl = pl.ds(c * CHUNK, CHUNK)
        b, pb, sb = bufs[st]
        pltpu.async_copy(word_hbm.at[widx_v.at[isl]], b, sem_g[st])
        pltpu.async_copy(post_hbm.at[pidx_v.at[isl]], pb, sem_g[st])

        @pl.when(needs_small(c))
        def _():
            pltpu.async_copy(small_hbm.at[sidx_v.at[isl]], sb, sem_g[st])

    def drain_gathers(c, st):
        b = bufs[st][0]
        for _ in range(2):
            pltpu.make_async_copy(word_hbm.at[pl.ds(0, CHUNK)], b, sem_g[st]).wait()

        @pl.when(needs_small(c))
        def _():
            pltpu.make_async_copy(word_hbm.at[pl.ds(0, CHUNK)], b, sem_g[st]).wait()

    def drain_store(st):
        pltpu.make_async_copy(bufs[st][0], out_hbm.at[pl.ds(0, CHUNK)],
                              sem_s[st]).wait()

    for k in range(AHEAD):
        issue_gathers(k, k)

    def chunk_group(cc, _):
        for j in range(NSETS):
            c = cc * NSETS + j
            b, pb, sb = bufs[j]
            nst = (j + AHEAD) % NSETS

            @pl.when(c + AHEAD < NCHUNK)
            def _prefetch():
                @pl.when(c >= AHEAD)
                def _():
                    drain_store(nst)
                issue_gathers(c + AHEAD, nst)

            drain_gathers(c, j)

            @pl.when(needs_small(c))
            def _combine3():
                def row_body(r, _):
                    for k in range(EMBED // LANES):
                        sl = pl.ds(k * LANES, LANES)
                        b[r, sl] = b[r, sl] + pb[r, sl] + sb[r, sl]
                    return _

                lax.fori_loop(0, CHUNK, row_body, None)

            @pl.when(jnp.logical_not(needs_small(c)))
            def _combine2():
                def row_body(r, _):
                    for k in range(EMBED // LANES):
                        sl = pl.ds(k * LANES, LANES)
                        b[r, sl] = b[r, sl] + pb[r, sl]
                    return _

                lax.fori_loop(0, CHUNK, row_body, None)

            pltpu.async_copy(b, out_hbm.at[pl.ds(base + c * CHUNK, CHUNK)],
                             sem_s[j])
        return _

    lax.fori_loop(0, NCHUNK // NSETS, chunk_group, None)
    for k in range(NSETS):
        drain_store(k)


@jax.jit
def _run(tok_flat, pos_flat, word_table, pos_table, small_table):
    mesh = plsc.VectorSubcoreMesh(core_axis_name="c", subcore_axis_name="s")
    f = pl.kernel(
        _sc_body,
        out_type=jax.ShapeDtypeStruct((NUM_TOK, EMBED), jnp.float32),
        mesh=mesh,
        scratch_types=[pltpu.VMEM((T_PER_W,), jnp.int32)] * 5
        + [pltpu.SMEM((T_PER_W // LANES,), jnp.int32)]
        + [pltpu.VMEM((CHUNK, EMBED), jnp.float32)] * (3 * NSETS)
        + [pltpu.SemaphoreType.DMA] * (2 * NSETS),
    )
    return f(tok_flat, pos_flat, word_table, pos_table, small_table)


def kernel(tokens, position_ids, word_table, pos_table, bbox_table, ocr_table):
    b, s = tokens.shape
    tok_flat = tokens.reshape(-1).astype(jnp.int32)
    pos_flat = position_ids.reshape(-1).astype(jnp.int32)
    nsm = bbox_table.shape[0] + ocr_table.shape[0]
    corr = word_table[:nsm] + pos_table[:nsm]
    small_table = jnp.concatenate(
        [jnp.concatenate([bbox_table, ocr_table], axis=0) - corr,
         jnp.zeros((NZ, EMBED), jnp.float32)], axis=0)
    out = _run(tok_flat, pos_flat, word_table, pos_table, small_table)
    return out.reshape(b, s, EMBED)


# speedup vs baseline: 1.1569x; 1.1569x over previous
import jax
import jax.numpy as jnp
from jax import lax
from jax.experimental import pallas as pl
from jax.experimental.pallas import tpu as pltpu
from jax.experimental.pallas import tpu_sc as plsc

IPT_VOCAB = 100000
EMBED = 1024
NUM_TOK = 4 * 4096
LANES = 16

NC, NS = 2, 16
NW = NC * NS
T_PER_W = NUM_TOK // NW
CHUNK = 16
NCHUNK = T_PER_W // CHUNK
NSETS = 2
AHEAD = 1
ZROW = 1026
NZ = 512


def _sc_body(tok_hbm, posid_hbm, word_hbm, post_hbm, small_hbm, out_hbm,
             tok_v, pos_v, widx_v, pidx_v, sidx_v, flag_s,
             *bufs_and_sems):
    bufs = tuple((bufs_and_sems[3 * k], bufs_and_sems[3 * k + 1],
                  bufs_and_sems[3 * k + 2]) for k in range(NSETS))
    sem_g = bufs_and_sems[3 * NSETS:3 * NSETS + NSETS]
    sem_s = bufs_and_sems[3 * NSETS + NSETS:]

    wid = lax.axis_index("s") * NC + lax.axis_index("c")
    base = wid * T_PER_W

    pltpu.sync_copy(tok_hbm.at[pl.ds(base, T_PER_W)], tok_v)
    pltpu.sync_copy(posid_hbm.at[pl.ds(base, T_PER_W)], pos_v)

    def idx_body(i, _):
        sl = pl.ds(i * LANES, LANES)
        t = tok_v[sl]
        p = pos_v[sl]
        ipt = t < IPT_VOCAB
        zrow = ZROW + (i % (NZ // LANES)) * LANES + lax.iota(jnp.int32, 16)
        nonipt_row = t - IPT_VOCAB
        widx_v[sl] = jnp.where(ipt, t, nonipt_row)
        pidx_v[sl] = jnp.where(ipt, p, nonipt_row)
        sidx_v[sl] = jnp.where(ipt, zrow, nonipt_row)
        m = t[0]
        for k in range(1, LANES):
            m = jnp.maximum(m, t[k])
        flag_s[i] = m
        return _

    lax.fori_loop(0, T_PER_W // LANES, idx_body, None)

    def needs_small(c):
        return flag_s[c * CHUNK // LANES] >= IPT_VOCAB

    def issue_pos_small(c, st):
        isl = pl.ds(c * CHUNK, CHUNK)
        _, pb, sb = bufs[st]
        pltpu.async_copy(post_hbm.at[pidx_v.at[isl]], pb, sem_g[st])

        @pl.when(needs_small(c))
        def _():
            pltpu.async_copy(small_hbm.at[sidx_v.at[isl]], sb, sem_g[st])

    def issue_word(c, st):
        isl = pl.ds(c * CHUNK, CHUNK)
        pltpu.async_copy(word_hbm.at[widx_v.at[isl]], bufs[st][0], sem_g[st])

    def issue_gathers(c, st):
        issue_pos_small(c, st)
        issue_word(c, st)

    def drain_gathers(c, st):
        b = bufs[st][0]
        for _ in range(2):
            pltpu.make_async_copy(word_hbm.at[pl.ds(0, CHUNK)], b, sem_g[st]).wait()

        @pl.when(needs_small(c))
        def _():
            pltpu.make_async_copy(word_hbm.at[pl.ds(0, CHUNK)], b, sem_g[st]).wait()

    def drain_store(st):
        pltpu.make_async_copy(bufs[st][0], out_hbm.at[pl.ds(0, CHUNK)],
                              sem_s[st]).wait()

    for k in range(AHEAD):
        issue_gathers(k, k)

    def chunk_group(cc, _):
        for j in range(NSETS):
            c = cc * NSETS + j
            b, pb, sb = bufs[j]
            nst = (j + AHEAD) % NSETS

            @pl.when(c + AHEAD < NCHUNK)
            def _prefetch():
                issue_pos_small(c + AHEAD, nst)

                @pl.when(c >= AHEAD)
                def _():
                    drain_store(nst)
                issue_word(c + AHEAD, nst)

            drain_gathers(c, j)

            @pl.when(needs_small(c))
            def _combine3():
                def row_body(r, _):
                    for k in range(EMBED // LANES):
                        sl = pl.ds(k * LANES, LANES)
                        b[r, sl] = b[r, sl] + pb[r, sl] + sb[r, sl]
                    return _

                lax.fori_loop(0, CHUNK, row_body, None)

            @pl.when(jnp.logical_not(needs_small(c)))
            def _combine2():
                def row_body(r, _):
                    for k in range(EMBED // LANES):
                        sl = pl.ds(k * LANES, LANES)
                        b[r, sl] = b[r, sl] + pb[r, sl]
                    return _

                lax.fori_loop(0, CHUNK, row_body, None)

            pltpu.async_copy(b, out_hbm.at[pl.ds(base + c * CHUNK, CHUNK)],
                             sem_s[j])
        return _

    lax.fori_loop(0, NCHUNK // NSETS, chunk_group, None)
    for k in range(NSETS):
        drain_store(k)


@jax.jit
def _run(tok_flat, pos_flat, word_table, pos_table, small_table):
    mesh = plsc.VectorSubcoreMesh(core_axis_name="c", subcore_axis_name="s")
    f = pl.kernel(
        _sc_body,
        out_type=jax.ShapeDtypeStruct((NUM_TOK, EMBED), jnp.float32),
        mesh=mesh,
        scratch_types=[pltpu.VMEM((T_PER_W,), jnp.int32)] * 5
        + [pltpu.SMEM((T_PER_W // LANES,), jnp.int32)]
        + [pltpu.VMEM((CHUNK, EMBED), jnp.float32)] * (3 * NSETS)
        + [pltpu.SemaphoreType.DMA] * (2 * NSETS),
    )
    return f(tok_flat, pos_flat, word_table, pos_table, small_table)


def kernel(tokens, position_ids, word_table, pos_table, bbox_table, ocr_table):
    b, s = tokens.shape
    tok_flat = tokens.reshape(-1).astype(jnp.int32)
    pos_flat = position_ids.reshape(-1).astype(jnp.int32)
    nsm = bbox_table.shape[0] + ocr_table.shape[0]
    corr = word_table[:nsm] + pos_table[:nsm]
    small_table = jnp.concatenate(
        [jnp.concatenate([bbox_table, ocr_table], axis=0) - corr,
         jnp.zeros((NZ, EMBED), jnp.float32)], axis=0)
    out = _run(tok_flat, pos_flat, word_table, pos_table, small_table)
    return out.reshape(b, s, EMBED)
